# Initial kernel scaffold; baseline (speedup 1.0000x reference)
import jax
import jax.numpy as jnp
from jax.experimental import pallas as pl


def kernel(X, word_table, pos_table):
    raise NotImplementedError("write your pallas kernel here")


# speedup vs baseline: 5.1679x; 5.1679x over previous
import functools

import jax
import jax.numpy as jnp
from jax import lax
from jax.experimental import pallas as pl
from jax.experimental.pallas import tpu as pltpu
from jax.experimental.pallas import tpu_sc as plsc

B = 16384
P = 12
V = 28
D = 24
NTOK = B * P
NW = 32
TOK_W = NTOK // NW
CHUNK = 1536
NCH = TOK_W // CHUNK
GRP = CHUNK // 16
LANES = 16


def _build_fused_body(word_ref, pos_ref, out_ref):
    out_ref[...] = pos_ref[...] + word_ref[...][None, :, :]


_build_fused = pl.pallas_call(
    _build_fused_body,
    out_shape=jax.ShapeDtypeStruct((P, V, D), jnp.float32),
)

_sc_mesh = plsc.VectorSubcoreMesh(core_axis_name="c", subcore_axis_name="s")


@functools.partial(
    pl.kernel,
    mesh=_sc_mesh,
    compiler_params=pltpu.CompilerParams(needs_layout_passes=False),
    out_type=jax.ShapeDtypeStruct((NTOK * D,), jnp.float32),
    scratch_types=[
        pltpu.VMEM((P * V * D,), jnp.float32),
        pltpu.VMEM((TOK_W,), jnp.int32),
        pltpu.VMEM((CHUNK * D,), jnp.float32),
        pltpu.VMEM((CHUNK * D,), jnp.float32),
        pltpu.SemaphoreType.DMA,
        pltpu.SemaphoreType.DMA,
    ],
)
def _sc_embed(fused_hbm, x_hbm, out_hbm, fused_v, x_v, buf0, buf1, sem0, sem1):
    wid = lax.axis_index("s") * 2 + lax.axis_index("c")
    base = pl.multiple_of(wid * TOK_W, TOK_W)
    pltpu.sync_copy(fused_hbm, fused_v)
    pltpu.sync_copy(x_hbm.at[pl.ds(base, TOK_W)], x_v)

    lane = lax.iota(jnp.int32, LANES)
    lane_d = lane * D

    bufs = (buf0, buf1)
    sems = (sem0, sem1)

    def compute_chunk(c, buf):
        def group(g, carry):
            t = pl.multiple_of(c * CHUNK + g * LANES, LANES)
            xv = x_v[pl.ds(t, LANES)]
            lv = lax.rem(t + lane, P)
            row_d = (lv * V + xv) * D
            ob = g * (LANES * D)
            for dd in range(D):
                vals = plsc.load_gather(fused_v, [row_d + dd])
                plsc.store_scatter(buf, [lane_d + (ob + dd)], vals)
            return carry

        lax.fori_loop(0, GRP, group, 0)

    copies = []
    for c in range(NCH):
        bsel = c % 2
        if c >= 2:
            copies[c - 2].wait()
        compute_chunk(c, bufs[bsel])
        off = pl.multiple_of((base + c * CHUNK) * D, CHUNK * D)
        copies.append(
            pltpu.async_copy(bufs[bsel], out_hbm.at[pl.ds(off, CHUNK * D)],
                             sems[bsel]))
    copies[-2].wait()
    copies[-1].wait()


def kernel(X, word_table, pos_table):
    fused = _build_fused(word_table, pos_table[:, None, :])
    fused_flat = fused.reshape(P * V * D)
    x_flat = X.reshape(NTOK).astype(jnp.int32)
    out_flat = _sc_embed(fused_flat, x_flat)
    return out_flat.reshape(B, P, D)
